# baseline (speedup 1.0000x reference)
import functools

import jax
import jax.numpy as jnp
from jax.experimental import pallas as pl
from jax.experimental.pallas import tpu as pltpu

_TAPS = tuple((di, dj) for di in range(3) for dj in range(3))


def _inception_kernel(x_ref, wcat_ref, scat_ref, w12_ref, s12_ref,
                      w13_ref, s13_ref, w22_ref, s22_ref,
                      o_ref,
                      xpad_ref, xp_ref, p2_ref, x3a_ref, x3b_ref, x4_ref,
                      xma_ref, xmb_ref, p3_ref, p4_ref):
    f32 = jnp.float32
    B = x_ref.shape[0]
    M1 = B * 35 * 40
    M3 = B * 17 * 24

    xin = jnp.transpose(x_ref[...], (0, 2, 3, 1))
    xpad_ref[:, :, 0:35, :] = xin
    xb = xpad_ref[...].reshape(M1, 192).astype(jnp.bfloat16)
    r = jnp.dot(xb, wcat_ref[...], preferred_element_type=f32)
    r = jnp.maximum(r + scat_ref[...], 0.0)
    h1 = r[:, :64].reshape(B, 35, 40, 64)
    x4_ref[...] = jnp.pad(r[:, 64:], ((0, 0), (0, 64))).reshape(
        B, 35, 40, 128)

    xma_ref[...] = xin[:, :, :, :128]
    xmb_ref[...] = jnp.pad(xin[:, :, :, 128:], ((0, 0),) * 3 + ((0, 64),))

    xp_ref[...] = jnp.zeros(xp_ref.shape, f32)
    xp_ref[:, 1:36, 1:36, :] = h1[:, :, :35, :]
    for t, (di, dj) in enumerate(_TAPS):
        slab = xp_ref[:, di:di + 35, dj:dj + 40, :]
        p2_ref[:, 64 * t:64 * (t + 1)] = slab.reshape(M1, 64).astype(
            jnp.bfloat16)
    h2 = jnp.dot(p2_ref[...], w12_ref[...], preferred_element_type=f32)
    h2 = jnp.maximum(h2 + s12_ref[...], 0.0)
    x3a_ref[...] = h2[:, :128].reshape(B, 35, 40, 128)
    x3b_ref[...] = jnp.pad(h2[:, 128:], ((0, 0), (0, 78))).reshape(
        B, 35, 40, 128)

    p3_ref[...] = jnp.zeros(p3_ref.shape, f32)
    p4_ref[...] = jnp.zeros(p4_ref.shape, f32)
    for t, (di, dj) in enumerate(_TAPS):
        p3_ref[:, :, 0:18, 256 * t:256 * t + 128] = (
            x3a_ref[:, di:di + 33:2, dj:dj + 35:2, :])
        p3_ref[:, :, 0:18, 256 * t + 128:256 * t + 256] = (
            x3b_ref[:, di:di + 33:2, dj:dj + 35:2, :])
        p4_ref[:, :, 0:18, 128 * t:128 * (t + 1)] = (
            x4_ref[:, di:di + 33:2, dj:dj + 35:2, :])

    o1 = jnp.dot(p3_ref[...].reshape(M3, 2304).astype(jnp.bfloat16),
                 w13_ref[...], preferred_element_type=f32)
    o1 = jnp.maximum(o1 + s13_ref[...], 0.0)
    o1s = o1.reshape(B, 17, 24, 178)[:, :, :17, :]

    o2 = jnp.dot(p4_ref[...].reshape(M3, 1152).astype(jnp.bfloat16),
                 w22_ref[...], preferred_element_type=f32)
    o2 = jnp.maximum(o2 + s22_ref[...], 0.0)
    o_ref[:, :, :, 178:480] = o2.reshape(B, 17, 24, 302)[:, :, :17, :]
    o_ref[:, :, :, 0:178] = o1s

    mpa = mpb = None
    for di, dj in _TAPS:
        wa = xma_ref[:, di:di + 33:2, dj:dj + 33:2, :]
        wb = xmb_ref[:, di:di + 33:2, dj:dj + 33:2, :]
        mpa = wa if mpa is None else jnp.maximum(mpa, wa)
        mpb = wb if mpb is None else jnp.maximum(mpb, wb)
    o_ref[:, :, :, 480:608] = mpa
    o_ref[:, :, :, 608:672] = mpb[:, :, :, :64]


@functools.partial(jax.jit, static_argnames=())
def kernel(x, b1_1_w, b1_1_s, b1_2_w, b1_2_s, b1_3_w, b1_3_s,
           b2_1_w, b2_1_s, b2_2_w, b2_2_s):
    N = x.shape[0]
    B = 2
    f32 = jnp.float32

    wcat = jnp.concatenate([b1_1_w, b2_1_w], axis=1)
    scat = jnp.concatenate([b1_1_s, b2_1_s]).reshape(1, 128).astype(f32)
    w13 = jnp.pad(b1_3_w.reshape(9, 178, 178),
                  ((0, 0), (0, 78), (0, 0))).reshape(2304, 178)
    w22 = jnp.pad(b2_2_w.reshape(9, 64, 302),
                  ((0, 0), (0, 64), (0, 0))).reshape(1152, 302)
    s12 = b1_2_s.reshape(1, 178).astype(f32)
    s13 = b1_3_s.reshape(1, 178).astype(f32)
    s22 = b2_2_s.reshape(1, 302).astype(f32)

    grid = (N // B,)
    out = pl.pallas_call(
        _inception_kernel,
        out_shape=jax.ShapeDtypeStruct((N, 17, 17, 672), f32),
        grid=grid,
        in_specs=[
            pl.BlockSpec((B, 192, 35, 35), lambda i: (i, 0, 0, 0)),
            pl.BlockSpec((192, 128), lambda i: (0, 0)),
            pl.BlockSpec((1, 128), lambda i: (0, 0)),
            pl.BlockSpec((576, 178), lambda i: (0, 0)),
            pl.BlockSpec((1, 178), lambda i: (0, 0)),
            pl.BlockSpec((2304, 178), lambda i: (0, 0)),
            pl.BlockSpec((1, 178), lambda i: (0, 0)),
            pl.BlockSpec((1152, 302), lambda i: (0, 0)),
            pl.BlockSpec((1, 302), lambda i: (0, 0)),
        ],
        out_specs=pl.BlockSpec((B, 17, 17, 672), lambda i: (i, 0, 0, 0)),
        scratch_shapes=[
            pltpu.VMEM((B, 35, 40, 192), f32),
            pltpu.VMEM((B, 37, 42, 64), f32),
            pltpu.VMEM((B * 35 * 40, 576), jnp.bfloat16),
            pltpu.VMEM((B, 35, 40, 128), f32),
            pltpu.VMEM((B, 35, 40, 128), f32),
            pltpu.VMEM((B, 35, 40, 128), f32),
            pltpu.VMEM((B, 35, 35, 128), f32),
            pltpu.VMEM((B, 35, 35, 128), f32),
            pltpu.VMEM((B, 17, 24, 2304), f32),
            pltpu.VMEM((B, 17, 24, 1152), f32),
        ],
        compiler_params=pltpu.CompilerParams(
            dimension_semantics=("parallel",)),
    )(x, wcat, scat, b1_2_w, s12, w13, s13, b2_2_w, s22)

    return jnp.transpose(out, (0, 3, 1, 2))


# speedup vs baseline: 1.5145x; 1.5145x over previous
import functools

import jax
import jax.numpy as jnp
from jax.experimental import pallas as pl
from jax.experimental.pallas import tpu as pltpu

_TAPS = tuple((di, dj) for di in range(3) for dj in range(3))


def _inception_kernel(x_ref, wcat_ref, scat_ref, w12_ref, s12_ref,
                      w13_ref, s13_ref, w22_ref, s22_ref,
                      o_ref,
                      xpad_ref, xp_ref, p2_ref, x3a_ref, x3b_ref, x4_ref,
                      xma_ref, xmb_ref, p3_ref, p4_ref):
    f32 = jnp.float32
    B = x_ref.shape[0]
    M1 = B * 35 * 40
    M3 = B * 17 * 24

    xin = x_ref[...]
    xpad_ref[:, :, 0:35, :] = xin
    xb = xpad_ref[...].reshape(M1, 192).astype(jnp.bfloat16)
    r = jnp.dot(xb, wcat_ref[...], preferred_element_type=f32)
    r = jnp.maximum(r + scat_ref[...], 0.0)
    h1 = r[:, :64].reshape(B, 35, 40, 64)
    x4_ref[...] = jnp.pad(r[:, 64:], ((0, 0), (0, 64))).reshape(
        B, 35, 40, 128)

    xma_ref[...] = xin[:, :, :, :128]
    xmb_ref[...] = jnp.pad(xin[:, :, :, 128:], ((0, 0),) * 3 + ((0, 64),))

    xp_ref[...] = jnp.zeros(xp_ref.shape, f32)
    xp_ref[:, 1:36, 1:36, :] = h1[:, :, :35, :]
    for t, (di, dj) in enumerate(_TAPS):
        slab = xp_ref[:, di:di + 35, dj:dj + 40, :]
        p2_ref[:, 64 * t:64 * (t + 1)] = slab.reshape(M1, 64).astype(
            jnp.bfloat16)
    h2 = jnp.dot(p2_ref[...], w12_ref[...], preferred_element_type=f32)
    h2 = jnp.maximum(h2 + s12_ref[...], 0.0)
    x3a_ref[...] = h2[:, :128].reshape(B, 35, 40, 128)
    x3b_ref[...] = jnp.pad(h2[:, 128:], ((0, 0), (0, 78))).reshape(
        B, 35, 40, 128)

    for t, (di, dj) in enumerate(_TAPS):
        p3_ref[:, :, 0:18, 256 * t:256 * t + 128] = (
            x3a_ref[:, di:di + 33:2, dj:dj + 35:2, :])
        p3_ref[:, :, 0:18, 256 * t + 128:256 * t + 256] = (
            x3b_ref[:, di:di + 33:2, dj:dj + 35:2, :])
        p4_ref[:, :, 0:18, 128 * t:128 * (t + 1)] = (
            x4_ref[:, di:di + 33:2, dj:dj + 35:2, :])

    o1 = jnp.dot(p3_ref[...].reshape(M3, 2304).astype(jnp.bfloat16),
                 w13_ref[...], preferred_element_type=f32)
    o1 = jnp.maximum(o1 + s13_ref[...], 0.0)
    o_ref[:, :, :, 0:178] = o1.reshape(B, 17, 24, 178)[:, :, :17, :]

    o2 = jnp.dot(p4_ref[...].reshape(M3, 1152).astype(jnp.bfloat16),
                 w22_ref[...], preferred_element_type=f32)
    o2 = jnp.maximum(o2 + s22_ref[...], 0.0)
    o_ref[:, :, :, 178:480] = o2.reshape(B, 17, 24, 302)[:, :, :17, :]

    mpa = mpb = None
    for di, dj in _TAPS:
        wa = xma_ref[:, di:di + 33:2, dj:dj + 33:2, :]
        wb = xmb_ref[:, di:di + 33:2, dj:dj + 33:2, :]
        mpa = wa if mpa is None else jnp.maximum(mpa, wa)
        mpb = wb if mpb is None else jnp.maximum(mpb, wb)
    o_ref[:, :, :, 480:608] = mpa
    o_ref[:, :, :, 608:672] = mpb[:, :, :, :64]


@functools.partial(jax.jit, static_argnames=())
def kernel(x, b1_1_w, b1_1_s, b1_2_w, b1_2_s, b1_3_w, b1_3_s,
           b2_1_w, b2_1_s, b2_2_w, b2_2_s):
    N = x.shape[0]
    B = 2
    f32 = jnp.float32

    x_nhwc = jnp.transpose(x, (0, 2, 3, 1))

    wcat = jnp.concatenate([b1_1_w, b2_1_w], axis=1)
    scat = jnp.concatenate([b1_1_s, b2_1_s]).reshape(1, 128).astype(f32)
    w13 = jnp.pad(b1_3_w.reshape(9, 178, 178),
                  ((0, 0), (0, 78), (0, 0))).reshape(2304, 178)
    w22 = jnp.pad(b2_2_w.reshape(9, 64, 302),
                  ((0, 0), (0, 64), (0, 0))).reshape(1152, 302)
    s12 = b1_2_s.reshape(1, 178).astype(f32)
    s13 = b1_3_s.reshape(1, 178).astype(f32)
    s22 = b2_2_s.reshape(1, 302).astype(f32)

    grid = (N // B,)
    out = pl.pallas_call(
        _inception_kernel,
        out_shape=jax.ShapeDtypeStruct((N, 17, 17, 672), f32),
        grid=grid,
        in_specs=[
            pl.BlockSpec((B, 35, 35, 192), lambda i: (i, 0, 0, 0)),
            pl.BlockSpec((192, 128), lambda i: (0, 0)),
            pl.BlockSpec((1, 128), lambda i: (0, 0)),
            pl.BlockSpec((576, 178), lambda i: (0, 0)),
            pl.BlockSpec((1, 178), lambda i: (0, 0)),
            pl.BlockSpec((2304, 178), lambda i: (0, 0)),
            pl.BlockSpec((1, 178), lambda i: (0, 0)),
            pl.BlockSpec((1152, 302), lambda i: (0, 0)),
            pl.BlockSpec((1, 302), lambda i: (0, 0)),
        ],
        out_specs=pl.BlockSpec((B, 17, 17, 672), lambda i: (i, 0, 0, 0)),
        scratch_shapes=[
            pltpu.VMEM((B, 35, 40, 192), f32),
            pltpu.VMEM((B, 37, 42, 64), f32),
            pltpu.VMEM((B * 35 * 40, 576), jnp.bfloat16),
            pltpu.VMEM((B, 35, 40, 128), f32),
            pltpu.VMEM((B, 35, 40, 128), f32),
            pltpu.VMEM((B, 35, 40, 128), f32),
            pltpu.VMEM((B, 35, 35, 128), f32),
            pltpu.VMEM((B, 35, 35, 128), f32),
            pltpu.VMEM((B, 17, 24, 2304), f32),
            pltpu.VMEM((B, 17, 24, 1152), f32),
        ],
        compiler_params=pltpu.CompilerParams(
            dimension_semantics=("parallel",)),
    )(x_nhwc, wcat, scat, b1_2_w, s12, w13, s13, b2_2_w, s22)

    return jnp.transpose(out, (0, 3, 1, 2))
